# Initial kernel scaffold; baseline (speedup 1.0000x reference)
import jax
import jax.numpy as jnp
from jax.experimental import pallas as pl


def kernel(x, token_table, pos_table):
    raise NotImplementedError("write your pallas kernel here")


# speedup vs baseline: 1.7340x; 1.7340x over previous
import functools

import jax
import jax.numpy as jnp
from jax import lax
from jax.experimental import pallas as pl
from jax.experimental.pallas import tpu as pltpu
from jax.experimental.pallas import tpu_sc as plsc

_VOCAB = 100000
_MAXLEN = 200
_EMBED = 64
_BATCH = 1024
_NC = 2
_NS = 16
_NW = _NC * _NS
_ROWS = _BATCH * _MAXLEN
_RPW = _ROWS // _NW
_CHUNK = 40
_NCHUNK = _RPW // _CHUNK
_LANES = 16


def _build():
    mesh = plsc.VectorSubcoreMesh(core_axis_name="c", subcore_axis_name="s")

    @functools.partial(
        pl.kernel,
        mesh=mesh,
        out_type=jax.ShapeDtypeStruct((_ROWS, _EMBED), jnp.float32),
        scratch_types=[
            pltpu.VMEM((_NCHUNK, _CHUNK), jnp.int32),
            pltpu.VMEM((_MAXLEN, _EMBED), jnp.float32),
            pltpu.VMEM((_CHUNK, _EMBED), jnp.float32),
            pltpu.SemaphoreType.DMA,
        ],
        compiler_params=pltpu.CompilerParams(use_tc_tiling_on_sc=False),
    )
    def k(x_hbm, tok_hbm, pos_hbm, out_hbm, idx_v, pos_v, rows_v, sem):
        wid = lax.axis_index("s") * _NC + lax.axis_index("c")
        base = wid * _RPW
        pltpu.sync_copy(pos_hbm, pos_v)
        pltpu.sync_copy(x_hbm.at[wid], idx_v)

        def chunk_body(j, carry):
            pltpu.async_copy(tok_hbm.at[idx_v.at[j]], rows_v, sem).wait()
            poff = (j % (_MAXLEN // _CHUNK)) * _CHUNK

            def row_body(l, c2):
                for t in range(_EMBED // _LANES):
                    sl = pl.ds(t * _LANES, _LANES)
                    rows_v[l, sl] = rows_v[l, sl] + pos_v[poff + l, sl]
                return c2

            lax.fori_loop(0, _CHUNK, row_body, 0, unroll=2)
            pltpu.sync_copy(rows_v, out_hbm.at[pl.ds(base + j * _CHUNK, _CHUNK)])
            return carry

        lax.fori_loop(0, _NCHUNK, chunk_body, 0)

    return k


_k = _build()


def kernel(x, token_table, pos_table):
    xw = x.reshape(_NW, _NCHUNK, _CHUNK).astype(jnp.int32)
    out = _k(xw, token_table, pos_table)
    return out.reshape(_BATCH, _MAXLEN, _EMBED)


# speedup vs baseline: 2.5479x; 1.4694x over previous
import functools

import jax
import jax.numpy as jnp
from jax import lax
from jax.experimental import pallas as pl
from jax.experimental.pallas import tpu as pltpu
from jax.experimental.pallas import tpu_sc as plsc

_VOCAB = 100000
_MAXLEN = 200
_EMBED = 64
_BATCH = 1024
_NC = 2
_NS = 16
_NW = _NC * _NS
_ROWS = _BATCH * _MAXLEN
_RPW = _ROWS // _NW
_CHUNK = 128
_NCHUNK = _RPW // _CHUNK
_NBUF = 5
_LEAD = 3
_LANES = 16
_POSX = _MAXLEN + _CHUNK - 8


def _build():
    mesh = plsc.VectorSubcoreMesh(core_axis_name="c", subcore_axis_name="s")

    @functools.partial(
        pl.kernel,
        mesh=mesh,
        out_type=jax.ShapeDtypeStruct((_ROWS, _EMBED), jnp.float32),
        scratch_types=[
            pltpu.VMEM((_NCHUNK, _CHUNK), jnp.int32),
            pltpu.VMEM((_POSX, _EMBED), jnp.float32),
            pltpu.VMEM((_NBUF, _CHUNK, _EMBED), jnp.float32),
            pltpu.SemaphoreType.DMA((_NBUF,)),
            pltpu.SemaphoreType.DMA((_NBUF,)),
        ],
        compiler_params=pltpu.CompilerParams(use_tc_tiling_on_sc=False),
    )
    def k(x_hbm, tok_hbm, posx_hbm, out_hbm, idx_v, pos_v, rows_v, gsem, ssem):
        wid = lax.axis_index("s") * _NC + lax.axis_index("c")
        base = wid * _RPW
        pltpu.sync_copy(posx_hbm, pos_v)
        pltpu.sync_copy(x_hbm.at[wid], idx_v)

        def start_gather(j, b):
            pltpu.async_copy(tok_hbm.at[idx_v.at[j]], rows_v.at[b], gsem.at[b])

        def wait_gather(b):
            pltpu.make_async_copy(
                out_hbm.at[pl.ds(0, _CHUNK)], rows_v.at[b], gsem.at[b]
            ).wait()

        def start_store(jj, b):
            pltpu.async_copy(
                rows_v.at[b], out_hbm.at[pl.ds(base + jj * _CHUNK, _CHUNK)],
                ssem.at[b],
            )

        def wait_store(b):
            pltpu.make_async_copy(
                rows_v.at[b], out_hbm.at[pl.ds(0, _CHUNK)], ssem.at[b]
            ).wait()

        def add_pos(jj, b):
            poff = lax.rem(jj * _CHUNK, _MAXLEN)

            def row_body(r, c):
                for t in range(_EMBED // _LANES):
                    sl = pl.ds(t * _LANES, _LANES)
                    plsc.addupdate(rows_v.at[b, r, sl], pos_v[poff + r, sl])
                return c

            lax.fori_loop(0, _CHUNK, row_body, 0, unroll=8)

        for b in range(_LEAD):
            start_gather(b, b)

        def group(gi, c):
            g = gi * _NBUF
            for b in range(_NBUF):
                jj = g + b
                nxt = jj + _LEAD
                bk = (b + _LEAD) % _NBUF

                @pl.when(nxt < _NCHUNK)
                def _():
                    @pl.when(nxt >= _NBUF)
                    def _():
                        wait_store(bk)

                    start_gather(nxt, bk)

                wait_gather(b)
                add_pos(jj, b)
                start_store(jj, b)
            return c

        lax.fori_loop(0, _NCHUNK // _NBUF, group, 0)
        for b in range(_NBUF):
            wait_store(b)

    return k


_k = _build()


def kernel(x, token_table, pos_table):
    xw = x.reshape(_NW, _NCHUNK, _CHUNK).astype(jnp.int32)
    posx = jnp.concatenate([pos_table, pos_table[: _POSX - _MAXLEN]], axis=0)
    out = _k(xw, token_table, posx)
    return out.reshape(_BATCH, _MAXLEN, _EMBED)
